# baseline (speedup 1.0000x reference)
import functools

import jax
import jax.numpy as jnp
from jax import lax
from jax.experimental import pallas as pl
from jax.experimental.pallas import tpu as pltpu
from jax.experimental.pallas import tpu_sc as plsc


def _square_distance(src, dst):
    return (jnp.sum(src ** 2, -1)[:, :, None] + jnp.sum(dst ** 2, -1)[:, None, :]
            - 2.0 * jnp.einsum('bnc,bmc->bnm', src, dst))


def _index_points(points, idx):
    B = points.shape[0]
    batch = jnp.arange(B).reshape((B,) + (1,) * (idx.ndim - 1))
    return points[batch, idx]


def _farthest_point_sample(xyz, npoint):
    B, N, _ = xyz.shape

    def body(i, state):
        centroids, distance, farthest = state
        centroids = centroids.at[:, i].set(farthest)
        centroid = jnp.take_along_axis(xyz, farthest[:, None, None], axis=1)
        dist = jnp.sum((xyz - centroid) ** 2, -1)
        distance = jnp.minimum(distance, dist)
        farthest = jnp.argmax(distance, -1).astype(jnp.int32)
        return centroids, distance, farthest

    init = (jnp.zeros((B, npoint), jnp.int32), jnp.full((B, N), 1e10, jnp.float32), jnp.zeros((B,), jnp.int32))
    centroids, _, _ = jax.lax.fori_loop(0, npoint, body, init)
    return centroids


def _query_ball_point(radius, nsample, xyz, new_xyz):
    N = xyz.shape[1]
    sqrdists = _square_distance(new_xyz, xyz)
    gi = jnp.where(sqrdists > radius ** 2, N, jnp.arange(N, dtype=jnp.int32)[None, None, :])
    gi = -jax.lax.top_k(-gi, nsample)[0]
    first = gi[:, :, :1]
    gi = jnp.where(gi == N, first, gi)
    gi = jnp.where(gi == N, 0, gi)
    return gi


def _mlp_apply(x, layers):
    for (W, b, g, be) in layers:
        x = x @ W + b
        x = g * x + be
        x = jax.nn.relu(x)
    return x


def _sa1_scale_body(p_ref, x_ref, X1_ref, w0_ref, w1_ref, b1_ref,
                    w2_ref, b2_ref, w3_ref, b3_ref, out_ref, *, K, SB):
    p = p_ref[0, 0]
    xg = x_ref[0, 0]
    Xs = X1_ref[0]
    d = xg - Xs
    h = (p[:, :, None] * w0_ref[0][None, None, :]
         + d[:, :, None] * w1_ref[0][None, None, :]
         + b1_ref[0][None, None, :])
    h = jax.nn.relu(h).reshape(SB * K, -1)
    h = jax.nn.relu(jnp.dot(h, w2_ref[...], preferred_element_type=jnp.float32) + b2_ref[...])
    h = jax.nn.relu(jnp.dot(h, w3_ref[...], preferred_element_type=jnp.float32) + b3_ref[...])
    out_ref[0] = jnp.max(h.reshape(SB, K, -1), axis=1)


def _sa1_scale(G, X1, fl, K):
    _, B, S, _ = G.shape
    B3 = 3 * B
    SB = 64
    (W1, b1), (W2, b2), (W3, b3) = fl
    c1, c2, c3 = W1.shape[1], W2.shape[1], W3.shape[1]
    body = functools.partial(_sa1_scale_body, K=K, SB=SB)
    return pl.pallas_call(
        body,
        grid=(B3, S // SB),
        in_specs=[
            pl.BlockSpec((1, 1, SB, K), lambda i, j: (i % 3, i // 3, j, 0)),
            pl.BlockSpec((1, 1, SB, K), lambda i, j: (3 + i % 3, i // 3, j, 0)),
            pl.BlockSpec((1, SB, 1), lambda i, j: (i, j, 0)),
            pl.BlockSpec((1, c1), lambda i, j: (0, 0)),
            pl.BlockSpec((1, c1), lambda i, j: (0, 0)),
            pl.BlockSpec((1, c1), lambda i, j: (0, 0)),
            pl.BlockSpec((c1, c2), lambda i, j: (0, 0)),
            pl.BlockSpec((1, c2), lambda i, j: (0, 0)),
            pl.BlockSpec((c2, c3), lambda i, j: (0, 0)),
            pl.BlockSpec((1, c3), lambda i, j: (0, 0)),
        ],
        out_specs=pl.BlockSpec((1, SB, c3), lambda i, j: (i, j, 0)),
        out_shape=jax.ShapeDtypeStruct((B3, S, c3), jnp.float32),
    )(G, G, X1, W1[0:1], W1[1:2], b1.reshape(1, c1),
      W2, b2.reshape(1, c2), W3, b3.reshape(1, c3))


def _sa1_msg(xyz, points, npoint, radii, nsamples, scale_params):
    B, _, N = xyz.shape
    S = npoint
    nx1 = _fps_coords(jnp.transpose(xyz, (1, 0, 2)), npoint)
    xyz_t = jnp.transpose(xyz, (0, 2, 1))
    new_xyz_t = jnp.transpose(nx1, (1, 2, 0))
    arr6 = jnp.concatenate([points.reshape(B, 3, N), xyz], axis=1)
    X1 = jnp.transpose(nx1, (1, 0, 2)).reshape(3 * B, S, 1)
    outs = []
    for radius, K, mlp_p in zip(radii, nsamples, scale_params):
        idx = _query_ball_point(radius, K, xyz_t, new_xyz_t)
        G = jnp.take_along_axis(arr6, idx.reshape(B, 1, S * K), axis=2)
        G = jnp.transpose(G.reshape(B, 6, S, K), (1, 0, 2, 3))
        outs.append(_sa1_scale(G, X1, _fold_bn(mlp_p), K))
    new_points = jnp.transpose(jnp.concatenate(outs, -1), (0, 2, 1))
    return jnp.transpose(new_xyz_t, (0, 2, 1)), nx1, new_points


def _fold_bn(layers):
    return [(W * g[None, :], b * g + be) for (W, b, g, be) in layers]


def _sa2_scale_body(pts_ref, x1_ref, X2_ref, idx_ref,
                    w1a_ref, w1b_ref, b1_ref, w2_ref, b2_ref, w3_ref, b3_ref,
                    out_ref, *, K, SB, N1):
    pts = pts_ref[0]
    xcol = jnp.transpose(x1_ref[0])
    T = jnp.dot(pts, w1a_ref[...], preferred_element_type=jnp.float32) \
        + xcol * w1b_ref[...]
    Xs = X2_ref[0]
    C = b1_ref[...] - Xs * w1b_ref[...]
    idxv = idx_ref[0]
    iota3 = jax.lax.broadcasted_iota(jnp.int32, (SB, K, N1), 2)
    OH = (idxv[:, :, None] == iota3).astype(jnp.float32).reshape(SB * K, N1)
    h = jnp.dot(OH, T, preferred_element_type=jnp.float32)
    h = jax.nn.relu(h.reshape(SB, K, -1) + C[:, None, :]).reshape(SB * K, -1)
    h = jax.nn.relu(jnp.dot(h, w2_ref[...], preferred_element_type=jnp.float32) + b2_ref[...])
    h = jax.nn.relu(jnp.dot(h, w3_ref[...], preferred_element_type=jnp.float32) + b3_ref[...])
    out_ref[0] = jnp.max(h.reshape(SB, K, -1), axis=1)


def _sa2_scale(pts_t, x1, X2, idx, fl, K):
    B3, N1, CIN = pts_t.shape
    S2 = X2.shape[1]
    SB = 32
    (W1, b1), (W2, b2), (W3, b3) = fl
    c1, c2, c3 = W1.shape[1], W2.shape[1], W3.shape[1]
    w1a, w1b = W1[:CIN], W1[CIN:].reshape(1, c1)
    body = functools.partial(_sa2_scale_body, K=K, SB=SB, N1=N1)
    return pl.pallas_call(
        body,
        grid=(B3, S2 // SB),
        in_specs=[
            pl.BlockSpec((1, N1, CIN), lambda i, j: (i, 0, 0)),
            pl.BlockSpec((1, 1, N1), lambda i, j: (i, 0, 0)),
            pl.BlockSpec((1, SB, 1), lambda i, j: (i, j, 0)),
            pl.BlockSpec((1, SB, K), lambda i, j: (i // 3, j, 0)),
            pl.BlockSpec((CIN, c1), lambda i, j: (0, 0)),
            pl.BlockSpec((1, c1), lambda i, j: (0, 0)),
            pl.BlockSpec((1, c1), lambda i, j: (0, 0)),
            pl.BlockSpec((c1, c2), lambda i, j: (0, 0)),
            pl.BlockSpec((1, c2), lambda i, j: (0, 0)),
            pl.BlockSpec((c2, c3), lambda i, j: (0, 0)),
            pl.BlockSpec((1, c3), lambda i, j: (0, 0)),
        ],
        out_specs=pl.BlockSpec((1, SB, c3), lambda i, j: (i, j, 0)),
        out_shape=jax.ShapeDtypeStruct((B3, S2, c3), jnp.float32),
    )(pts_t, x1, X2, idx, w1a, w1b, b1.reshape(1, c1), W2, b2.reshape(1, c2),
      W3, b3.reshape(1, c3))


def _sa2_msg(nx1, l1_points, npoint, radii, nsamples, scale_params):
    _, B, N1 = nx1.shape
    nx2 = _fps_coords(nx1, npoint)
    xyz_t = jnp.transpose(nx1, (1, 2, 0))
    new_xyz_t = jnp.transpose(nx2, (1, 2, 0))
    pts_t = jnp.transpose(l1_points, (0, 2, 1))
    x1 = jnp.transpose(nx1, (1, 0, 2)).reshape(B * 3, 1, N1)
    X2 = jnp.transpose(nx2, (1, 0, 2)).reshape(B * 3, npoint, 1)
    outs = []
    for radius, K, mlp_p in zip(radii, nsamples, scale_params):
        idx = _query_ball_point(radius, K, xyz_t, new_xyz_t)
        outs.append(_sa2_scale(pts_t, x1, X2, idx, _fold_bn(mlp_p), K))
    new_points = jnp.transpose(jnp.concatenate(outs, -1), (0, 2, 1))
    return jnp.transpose(new_xyz_t, (0, 2, 1)), nx2, new_points


def _sa_groupall_q(xyz, points, mlp_p):
    B, _, N = xyz.shape
    B3 = B * 3
    xyz_c = xyz.reshape(B3, N)[:, :, None]
    points_t = jnp.transpose(points, (0, 2, 1))
    feat = jnp.concatenate([points_t, xyz_c], axis=-1)[:, None, :, :]
    feat = _mlp_apply(feat, mlp_p)
    new_points = jnp.transpose(jnp.max(feat, axis=2), (0, 2, 1))
    new_xyz = jnp.zeros((B, 3, 1), xyz.dtype)
    return new_xyz, new_points


def _fp_q(xyz1, xyz2, points1, points2, mlp_p):
    B, _, N = xyz1.shape
    S = xyz2.shape[2]
    B3 = B * 3
    points2_t = jnp.transpose(points2, (0, 2, 1))
    if S == 1:
        interp = jnp.broadcast_to(points2_t, (B3, N, points2_t.shape[2]))
    else:
        xyz1_t = jnp.transpose(xyz1, (0, 2, 1))
        xyz2_t = jnp.transpose(xyz2, (0, 2, 1))
        d = _square_distance(xyz1_t, xyz2_t)
        negd, idx = jax.lax.top_k(-d, 3)
        dist = jnp.maximum(-negd, 0.0)
        w = 1.0 / (dist + 1e-8)
        w = w / jnp.sum(w, -1, keepdims=True)
        idx3 = jnp.repeat(idx, 3, axis=0)
        w3 = jnp.repeat(w, 3, axis=0)
        interp = jnp.sum(_index_points(points2_t, idx3) * w3[..., None], axis=2)
    points1_t = jnp.transpose(points1, (0, 2, 1))
    new = jnp.concatenate([points1_t, interp], axis=-1)
    new = _mlp_apply(new, mlp_p)
    return jnp.transpose(new, (0, 2, 1))


def _fps_body(xyz_ref, out_ref, *, S):
    X = xyz_ref[0]
    Y = xyz_ref[1]
    Z = xyz_ref[2]
    B, N = X.shape
    iota = jax.lax.broadcasted_iota(jnp.int32, (B, N), 1)
    iota_s = jax.lax.broadcasted_iota(jnp.int32, (B, S), 1)

    def body(i, carry):
        distance, onehot, ax, ay, az = carry
        cx = jnp.sum(X * onehot, axis=1, keepdims=True)
        cy = jnp.sum(Y * onehot, axis=1, keepdims=True)
        cz = jnp.sum(Z * onehot, axis=1, keepdims=True)
        sel = (iota_s == i).astype(jnp.float32)
        ax = ax + cx * sel
        ay = ay + cy * sel
        az = az + cz * sel
        d = (X - cx) ** 2 + (Y - cy) ** 2 + (Z - cz) ** 2
        distance = jnp.minimum(distance, d)
        m = jnp.max(distance, axis=1, keepdims=True)
        fidx = jnp.min(jnp.where(distance == m, iota, N), axis=1, keepdims=True)
        onehot = (iota == fidx).astype(jnp.float32)
        return distance, onehot, ax, ay, az

    zero_s = jnp.zeros((B, S), jnp.float32)
    _, _, ax, ay, az = jax.lax.fori_loop(
        0, S, body,
        (jnp.full((B, N), 1e10, jnp.float32), (iota == 0).astype(jnp.float32),
         zero_s, zero_s, zero_s),
    )
    out_ref[0] = ax
    out_ref[1] = ay
    out_ref[2] = az


def _fps_coords(xyz3, S):
    _, B, N = xyz3.shape
    return pl.pallas_call(
        functools.partial(_fps_body, S=S),
        in_specs=[pl.BlockSpec((3, B, N), lambda: (0, 0, 0))],
        out_specs=pl.BlockSpec((3, B, S), lambda: (0, 0, 0)),
        out_shape=jax.ShapeDtypeStruct((3, B, S), jnp.float32),
    )(xyz3)


def _head_body(feat_ref, w1_ref, b1_ref, g1_ref, be1_ref, w2_ref, b2_ref, out_ref):
    f = feat_ref[...]
    merged = jnp.sqrt(f[0] ** 2 + f[1] ** 2 + f[2] ** 2 + 1e-12)
    h = jnp.dot(merged, w1_ref[...], preferred_element_type=jnp.float32) + b1_ref[...]
    h = jax.nn.relu(g1_ref[...] * h + be1_ref[...])
    logits = jnp.dot(h, w2_ref[...], preferred_element_type=jnp.float32) + b2_ref[...]
    m = jnp.max(logits, axis=-1, keepdims=True)
    lse = jnp.log(jnp.sum(jnp.exp(logits - m), axis=-1, keepdims=True))
    out_ref[0] = logits - m - lse


def _head(l0_feat, W1, b1, g1, be1, W2, b2):
    B3, C, N = l0_feat.shape
    B = B3 // 3
    NB = 512
    feat_t = jnp.transpose(l0_feat, (0, 2, 1))
    NC = W2.shape[1]
    out = pl.pallas_call(
        _head_body,
        grid=(B, N // NB),
        in_specs=[
            pl.BlockSpec((3, NB, C), lambda b, j: (b, j, 0)),
            pl.BlockSpec((C, C), lambda b, j: (0, 0)),
            pl.BlockSpec((1, C), lambda b, j: (0, 0)),
            pl.BlockSpec((1, C), lambda b, j: (0, 0)),
            pl.BlockSpec((1, C), lambda b, j: (0, 0)),
            pl.BlockSpec((C, NC), lambda b, j: (0, 0)),
            pl.BlockSpec((1, NC), lambda b, j: (0, 0)),
        ],
        out_specs=pl.BlockSpec((1, NB, NC), lambda b, j: (b, j, 0)),
        out_shape=jax.ShapeDtypeStruct((B, N, NC), jnp.float32),
    )(feat_t, W1, b1.reshape(1, C), g1.reshape(1, C), be1.reshape(1, C),
      W2, b2.reshape(1, NC))
    return out


def kernel(xyz, norm_plt, cls_label, params):
    B, C, N = xyz.shape
    l0_xyz = xyz
    l0_points = norm_plt.reshape(B * 3, -1, N)
    l1_xyz, nx1, l1_points = _sa1_msg(l0_xyz, l0_points, 512, [0.1, 0.2, 0.4], [32, 64, 128], params['sa1'])
    l2_xyz, _, l2_points = _sa2_msg(nx1, l1_points, 128, [0.4, 0.8], [64, 128], params['sa2'])
    l3_xyz, l3_points = _sa_groupall_q(l2_xyz, l2_points, params['sa3'])
    l2_points = _fp_q(l2_xyz, l3_xyz, l2_points, l3_points, params['fp3'])
    l1_points = _fp_q(l1_xyz, l2_xyz, l1_points, l2_points, params['fp2'])
    cls_oh = jnp.tile(cls_label.reshape(B, 1, 16, 1), (1, 3, 1, N)).reshape(B * 3, 16, N)
    l0_cat = jnp.concatenate([cls_oh, l0_xyz.reshape(B * 3, 1, N), l0_points], axis=1)
    l0_feat = _fp_q(l0_xyz, l1_xyz, l0_cat, l1_points, params['fp1'])
    W1, b1 = params['conv1']
    g1, be1 = params['bn1']
    W2, b2 = params['conv2']
    return _head(l0_feat.reshape(B * 3, 128, N), W1, b1, g1, be1, W2, b2)


# speedup vs baseline: 1.0632x; 1.0632x over previous
import functools

import jax
import jax.numpy as jnp
from jax import lax
from jax.experimental import pallas as pl
from jax.experimental.pallas import tpu as pltpu
from jax.experimental.pallas import tpu_sc as plsc


def _square_distance(src, dst):
    return (jnp.sum(src ** 2, -1)[:, :, None] + jnp.sum(dst ** 2, -1)[:, None, :]
            - 2.0 * jnp.einsum('bnc,bmc->bnm', src, dst))


def _index_points(points, idx):
    B = points.shape[0]
    batch = jnp.arange(B).reshape((B,) + (1,) * (idx.ndim - 1))
    return points[batch, idx]


def _farthest_point_sample(xyz, npoint):
    B, N, _ = xyz.shape

    def body(i, state):
        centroids, distance, farthest = state
        centroids = centroids.at[:, i].set(farthest)
        centroid = jnp.take_along_axis(xyz, farthest[:, None, None], axis=1)
        dist = jnp.sum((xyz - centroid) ** 2, -1)
        distance = jnp.minimum(distance, dist)
        farthest = jnp.argmax(distance, -1).astype(jnp.int32)
        return centroids, distance, farthest

    init = (jnp.zeros((B, npoint), jnp.int32), jnp.full((B, N), 1e10, jnp.float32), jnp.zeros((B,), jnp.int32))
    centroids, _, _ = jax.lax.fori_loop(0, npoint, body, init)
    return centroids


def _query_ball_point(radius, nsample, xyz, new_xyz):
    N = xyz.shape[1]
    sqrdists = _square_distance(new_xyz, xyz)
    gi = jnp.where(sqrdists > radius ** 2, N, jnp.arange(N, dtype=jnp.int32)[None, None, :])
    gi = -jax.lax.top_k(-gi, nsample)[0]
    first = gi[:, :, :1]
    gi = jnp.where(gi == N, first, gi)
    gi = jnp.where(gi == N, 0, gi)
    return gi


def _mlp_apply(x, layers):
    for (W, b, g, be) in layers:
        x = x @ W + b
        x = g * x + be
        x = jax.nn.relu(x)
    return x


def _sa1_scale_body(p_ref, x_ref, X1_ref, w0_ref, w1_ref, b1_ref,
                    w2_ref, b2_ref, w3_ref, b3_ref, out_ref, *, K, SB):
    p = p_ref[0, 0]
    xg = x_ref[0, 0]
    Xs = X1_ref[0]
    d = xg - Xs
    h = (p[:, :, None] * w0_ref[0][None, None, :]
         + d[:, :, None] * w1_ref[0][None, None, :]
         + b1_ref[0][None, None, :])
    h = jax.nn.relu(h).reshape(SB * K, -1)
    h = jax.nn.relu(jnp.dot(h, w2_ref[...], preferred_element_type=jnp.float32) + b2_ref[...])
    h = jax.nn.relu(jnp.dot(h, w3_ref[...], preferred_element_type=jnp.float32) + b3_ref[...])
    out_ref[0] = jnp.max(h.reshape(SB, K, -1), axis=1)


def _sa1_scale(G, X1, fl, K):
    _, B, S, _ = G.shape
    B3 = 3 * B
    SB = 64
    (W1, b1), (W2, b2), (W3, b3) = fl
    c1, c2, c3 = W1.shape[1], W2.shape[1], W3.shape[1]
    body = functools.partial(_sa1_scale_body, K=K, SB=SB)
    return pl.pallas_call(
        body,
        grid=(B3, S // SB),
        in_specs=[
            pl.BlockSpec((1, 1, SB, K), lambda i, j: (i % 3, i // 3, j, 0)),
            pl.BlockSpec((1, 1, SB, K), lambda i, j: (3 + i % 3, i // 3, j, 0)),
            pl.BlockSpec((1, SB, 1), lambda i, j: (i, j, 0)),
            pl.BlockSpec((1, c1), lambda i, j: (0, 0)),
            pl.BlockSpec((1, c1), lambda i, j: (0, 0)),
            pl.BlockSpec((1, c1), lambda i, j: (0, 0)),
            pl.BlockSpec((c1, c2), lambda i, j: (0, 0)),
            pl.BlockSpec((1, c2), lambda i, j: (0, 0)),
            pl.BlockSpec((c2, c3), lambda i, j: (0, 0)),
            pl.BlockSpec((1, c3), lambda i, j: (0, 0)),
        ],
        out_specs=pl.BlockSpec((1, SB, c3), lambda i, j: (i, j, 0)),
        out_shape=jax.ShapeDtypeStruct((B3, S, c3), jnp.float32),
    )(G, G, X1, W1[0:1], W1[1:2], b1.reshape(1, c1),
      W2, b2.reshape(1, c2), W3, b3.reshape(1, c3))


def _sa1_msg(xyz, points, npoint, radii, nsamples, scale_params):
    B, _, N = xyz.shape
    S = npoint
    nx1 = _fps_coords(jnp.transpose(xyz, (1, 0, 2)), npoint)
    xyz_t = jnp.transpose(xyz, (0, 2, 1))
    new_xyz_t = jnp.transpose(nx1, (1, 2, 0))
    arr6 = jnp.concatenate([points.reshape(B, 3, N), xyz], axis=1)
    X1 = jnp.transpose(nx1, (1, 0, 2)).reshape(3 * B, S, 1)
    outs = []
    for radius, K, mlp_p in zip(radii, nsamples, scale_params):
        idx = _query_ball_point(radius, K, xyz_t, new_xyz_t)
        G = jnp.take_along_axis(arr6, idx.reshape(B, 1, S * K), axis=2)
        G = jnp.transpose(G.reshape(B, 6, S, K), (1, 0, 2, 3))
        outs.append(_sa1_scale(G, X1, _fold_bn(mlp_p), K))
    new_points = jnp.transpose(jnp.concatenate(outs, -1), (0, 2, 1))
    return jnp.transpose(new_xyz_t, (0, 2, 1)), nx1, new_points


def _fold_bn(layers):
    return [(W * g[None, :], b * g + be) for (W, b, g, be) in layers]


def _sa2_scale_body(pts_ref, x1_ref, X2_ref, idx_ref,
                    w1a_ref, w1b_ref, b1_ref, w2_ref, b2_ref, w3_ref, b3_ref,
                    out_ref, *, K, SB, N1):
    pts = pts_ref[0]
    xcol = jnp.transpose(x1_ref[0])
    T = jnp.dot(pts, w1a_ref[...], preferred_element_type=jnp.float32) \
        + xcol * w1b_ref[...]
    Xs = X2_ref[0]
    C = b1_ref[...] - Xs * w1b_ref[...]
    idxv = idx_ref[0]
    iota3 = jax.lax.broadcasted_iota(jnp.int32, (SB, K, N1), 2)
    OH = (idxv[:, :, None] == iota3).astype(jnp.float32).reshape(SB * K, N1)
    h = jnp.dot(OH, T, preferred_element_type=jnp.float32)
    h = jax.nn.relu(h.reshape(SB, K, -1) + C[:, None, :]).reshape(SB * K, -1)
    h = jax.nn.relu(jnp.dot(h, w2_ref[...], preferred_element_type=jnp.float32) + b2_ref[...])
    h = jax.nn.relu(jnp.dot(h, w3_ref[...], preferred_element_type=jnp.float32) + b3_ref[...])
    out_ref[0] = jnp.max(h.reshape(SB, K, -1), axis=1)


def _sa2_scale(pts_t, x1, X2, idx, fl, K):
    B3, N1, CIN = pts_t.shape
    S2 = X2.shape[1]
    SB = 32
    (W1, b1), (W2, b2), (W3, b3) = fl
    c1, c2, c3 = W1.shape[1], W2.shape[1], W3.shape[1]
    w1a, w1b = W1[:CIN], W1[CIN:].reshape(1, c1)
    body = functools.partial(_sa2_scale_body, K=K, SB=SB, N1=N1)
    return pl.pallas_call(
        body,
        grid=(B3, S2 // SB),
        in_specs=[
            pl.BlockSpec((1, N1, CIN), lambda i, j: (i, 0, 0)),
            pl.BlockSpec((1, 1, N1), lambda i, j: (i, 0, 0)),
            pl.BlockSpec((1, SB, 1), lambda i, j: (i, j, 0)),
            pl.BlockSpec((1, SB, K), lambda i, j: (i // 3, j, 0)),
            pl.BlockSpec((CIN, c1), lambda i, j: (0, 0)),
            pl.BlockSpec((1, c1), lambda i, j: (0, 0)),
            pl.BlockSpec((1, c1), lambda i, j: (0, 0)),
            pl.BlockSpec((c1, c2), lambda i, j: (0, 0)),
            pl.BlockSpec((1, c2), lambda i, j: (0, 0)),
            pl.BlockSpec((c2, c3), lambda i, j: (0, 0)),
            pl.BlockSpec((1, c3), lambda i, j: (0, 0)),
        ],
        out_specs=pl.BlockSpec((1, SB, c3), lambda i, j: (i, j, 0)),
        out_shape=jax.ShapeDtypeStruct((B3, S2, c3), jnp.float32),
    )(pts_t, x1, X2, idx, w1a, w1b, b1.reshape(1, c1), W2, b2.reshape(1, c2),
      W3, b3.reshape(1, c3))


def _sa2_msg(nx1, l1_points, npoint, radii, nsamples, scale_params):
    _, B, N1 = nx1.shape
    nx2 = _fps_coords(nx1, npoint)
    xyz_t = jnp.transpose(nx1, (1, 2, 0))
    new_xyz_t = jnp.transpose(nx2, (1, 2, 0))
    pts_t = jnp.transpose(l1_points, (0, 2, 1))
    x1 = jnp.transpose(nx1, (1, 0, 2)).reshape(B * 3, 1, N1)
    X2 = jnp.transpose(nx2, (1, 0, 2)).reshape(B * 3, npoint, 1)
    outs = []
    for radius, K, mlp_p in zip(radii, nsamples, scale_params):
        idx = _query_ball_point(radius, K, xyz_t, new_xyz_t)
        outs.append(_sa2_scale(pts_t, x1, X2, idx, _fold_bn(mlp_p), K))
    new_points = jnp.transpose(jnp.concatenate(outs, -1), (0, 2, 1))
    return jnp.transpose(new_xyz_t, (0, 2, 1)), nx2, new_points


def _sa_groupall_q(xyz, points, mlp_p):
    B, _, N = xyz.shape
    B3 = B * 3
    xyz_c = xyz.reshape(B3, N)[:, :, None]
    points_t = jnp.transpose(points, (0, 2, 1))
    feat = jnp.concatenate([points_t, xyz_c], axis=-1)[:, None, :, :]
    feat = _mlp_apply(feat, mlp_p)
    new_points = jnp.transpose(jnp.max(feat, axis=2), (0, 2, 1))
    new_xyz = jnp.zeros((B, 3, 1), xyz.dtype)
    return new_xyz, new_points


def _interp_body(p2_ref, idx_ref, w_ref, out_ref):
    p2 = p2_ref[0]
    S = p2.shape[0]
    idxv = idx_ref[0]
    wv = w_ref[0]
    NB = idxv.shape[1]
    iota = jax.lax.broadcasted_iota(jnp.int32, (NB, S), 1)
    OH = (wv[0][:, None] * (idxv[0][:, None] == iota)
          + wv[1][:, None] * (idxv[1][:, None] == iota)
          + wv[2][:, None] * (idxv[2][:, None] == iota))
    out_ref[0] = jnp.dot(OH, p2, preferred_element_type=jnp.float32)


def _interp(points2_t, idxT, wT, NB):
    B3, S, C = points2_t.shape
    N = idxT.shape[2]
    return pl.pallas_call(
        _interp_body,
        grid=(B3, N // NB),
        in_specs=[
            pl.BlockSpec((1, S, C), lambda i, j: (i, 0, 0)),
            pl.BlockSpec((1, 3, NB), lambda i, j: (i // 3, 0, j)),
            pl.BlockSpec((1, 3, NB), lambda i, j: (i // 3, 0, j)),
        ],
        out_specs=pl.BlockSpec((1, NB, C), lambda i, j: (i, j, 0)),
        out_shape=jax.ShapeDtypeStruct((B3, N, C), jnp.float32),
    )(points2_t, idxT, wT)


def _fp_q(xyz1, xyz2, points1, points2, mlp_p):
    B, _, N = xyz1.shape
    S = xyz2.shape[2]
    B3 = B * 3
    points2_t = jnp.transpose(points2, (0, 2, 1))
    if S == 1:
        interp = jnp.broadcast_to(points2_t, (B3, N, points2_t.shape[2]))
    else:
        xyz1_t = jnp.transpose(xyz1, (0, 2, 1))
        xyz2_t = jnp.transpose(xyz2, (0, 2, 1))
        d = _square_distance(xyz1_t, xyz2_t)
        negd, idx = jax.lax.top_k(-d, 3)
        dist = jnp.maximum(-negd, 0.0)
        w = 1.0 / (dist + 1e-8)
        w = w / jnp.sum(w, -1, keepdims=True)
        interp = _interp(points2_t, jnp.transpose(idx, (0, 2, 1)),
                         jnp.transpose(w, (0, 2, 1)), 256)
    points1_t = jnp.transpose(points1, (0, 2, 1))
    new = jnp.concatenate([points1_t, interp], axis=-1)
    new = _mlp_apply(new, mlp_p)
    return jnp.transpose(new, (0, 2, 1))


def _fps_body(xyz_ref, out_ref, *, S):
    X = xyz_ref[0]
    Y = xyz_ref[1]
    Z = xyz_ref[2]
    B, N = X.shape
    iota = jax.lax.broadcasted_iota(jnp.int32, (B, N), 1)
    iota_s = jax.lax.broadcasted_iota(jnp.int32, (B, S), 1)

    def body(i, carry):
        distance, onehot, ax, ay, az = carry
        cx = jnp.sum(X * onehot, axis=1, keepdims=True)
        cy = jnp.sum(Y * onehot, axis=1, keepdims=True)
        cz = jnp.sum(Z * onehot, axis=1, keepdims=True)
        sel = (iota_s == i).astype(jnp.float32)
        ax = ax + cx * sel
        ay = ay + cy * sel
        az = az + cz * sel
        d = (X - cx) ** 2 + (Y - cy) ** 2 + (Z - cz) ** 2
        distance = jnp.minimum(distance, d)
        m = jnp.max(distance, axis=1, keepdims=True)
        fidx = jnp.min(jnp.where(distance == m, iota, N), axis=1, keepdims=True)
        onehot = (iota == fidx).astype(jnp.float32)
        return distance, onehot, ax, ay, az

    zero_s = jnp.zeros((B, S), jnp.float32)
    _, _, ax, ay, az = jax.lax.fori_loop(
        0, S, body,
        (jnp.full((B, N), 1e10, jnp.float32), (iota == 0).astype(jnp.float32),
         zero_s, zero_s, zero_s),
    )
    out_ref[0] = ax
    out_ref[1] = ay
    out_ref[2] = az


def _fps_coords(xyz3, S):
    _, B, N = xyz3.shape
    return pl.pallas_call(
        functools.partial(_fps_body, S=S),
        in_specs=[pl.BlockSpec((3, B, N), lambda: (0, 0, 0))],
        out_specs=pl.BlockSpec((3, B, S), lambda: (0, 0, 0)),
        out_shape=jax.ShapeDtypeStruct((3, B, S), jnp.float32),
    )(xyz3)


def _head_body(feat_ref, w1_ref, b1_ref, g1_ref, be1_ref, w2_ref, b2_ref, out_ref):
    f = feat_ref[...]
    merged = jnp.sqrt(f[0] ** 2 + f[1] ** 2 + f[2] ** 2 + 1e-12)
    h = jnp.dot(merged, w1_ref[...], preferred_element_type=jnp.float32) + b1_ref[...]
    h = jax.nn.relu(g1_ref[...] * h + be1_ref[...])
    logits = jnp.dot(h, w2_ref[...], preferred_element_type=jnp.float32) + b2_ref[...]
    m = jnp.max(logits, axis=-1, keepdims=True)
    lse = jnp.log(jnp.sum(jnp.exp(logits - m), axis=-1, keepdims=True))
    out_ref[0] = logits - m - lse


def _head(l0_feat, W1, b1, g1, be1, W2, b2):
    B3, C, N = l0_feat.shape
    B = B3 // 3
    NB = 512
    feat_t = jnp.transpose(l0_feat, (0, 2, 1))
    NC = W2.shape[1]
    out = pl.pallas_call(
        _head_body,
        grid=(B, N // NB),
        in_specs=[
            pl.BlockSpec((3, NB, C), lambda b, j: (b, j, 0)),
            pl.BlockSpec((C, C), lambda b, j: (0, 0)),
            pl.BlockSpec((1, C), lambda b, j: (0, 0)),
            pl.BlockSpec((1, C), lambda b, j: (0, 0)),
            pl.BlockSpec((1, C), lambda b, j: (0, 0)),
            pl.BlockSpec((C, NC), lambda b, j: (0, 0)),
            pl.BlockSpec((1, NC), lambda b, j: (0, 0)),
        ],
        out_specs=pl.BlockSpec((1, NB, NC), lambda b, j: (b, j, 0)),
        out_shape=jax.ShapeDtypeStruct((B, N, NC), jnp.float32),
    )(feat_t, W1, b1.reshape(1, C), g1.reshape(1, C), be1.reshape(1, C),
      W2, b2.reshape(1, NC))
    return out


def kernel(xyz, norm_plt, cls_label, params):
    B, C, N = xyz.shape
    l0_xyz = xyz
    l0_points = norm_plt.reshape(B * 3, -1, N)
    l1_xyz, nx1, l1_points = _sa1_msg(l0_xyz, l0_points, 512, [0.1, 0.2, 0.4], [32, 64, 128], params['sa1'])
    l2_xyz, _, l2_points = _sa2_msg(nx1, l1_points, 128, [0.4, 0.8], [64, 128], params['sa2'])
    l3_xyz, l3_points = _sa_groupall_q(l2_xyz, l2_points, params['sa3'])
    l2_points = _fp_q(l2_xyz, l3_xyz, l2_points, l3_points, params['fp3'])
    l1_points = _fp_q(l1_xyz, l2_xyz, l1_points, l2_points, params['fp2'])
    cls_oh = jnp.tile(cls_label.reshape(B, 1, 16, 1), (1, 3, 1, N)).reshape(B * 3, 16, N)
    l0_cat = jnp.concatenate([cls_oh, l0_xyz.reshape(B * 3, 1, N), l0_points], axis=1)
    l0_feat = _fp_q(l0_xyz, l1_xyz, l0_cat, l1_points, params['fp1'])
    W1, b1 = params['conv1']
    g1, be1 = params['bn1']
    W2, b2 = params['conv2']
    return _head(l0_feat.reshape(B * 3, 128, N), W1, b1, g1, be1, W2, b2)
